# Initial kernel scaffold; baseline (speedup 1.0000x reference)
import jax
import jax.numpy as jnp
from jax.experimental import pallas as pl


def kernel(x, norm_w, Wg, bg, W1g, W1u, W2):
    raise NotImplementedError("write your pallas kernel here")


# speedup vs baseline: 1.8996x; 1.8996x over previous
import functools

import jax
import jax.numpy as jnp
from jax import lax
from jax.experimental import pallas as pl
from jax.experimental.pallas import tpu as pltpu
from jax.experimental.pallas import tpu_sc as plsc

T = 2048
D = 1024
F = 2048
E = 8
EPS = 1e-6

B = 576
NB = 7 + (T * 2 - 7 + B - 1) // B
P = NB * B
FT = 512
NF = F // FT

NC = 2
NS = 16
NW = NC * NS
TPW = T // NW
CH = 16


def _routing_body(x_ref, nw_ref, wg_ref, bg_ref,
                  t_ref, d1_ref, d2_ref, w1_ref, w2_ref, gid_ref, tgb_ref):
    x = x_ref[...]
    r = lax.rsqrt(jnp.mean(x * x, axis=1, keepdims=True) + EPS)
    t = x * r * nw_ref[...]
    t_ref[...] = t
    g = lax.dot_general(wg_ref[...], t, (((1,), (1,)), ((), ())),
                        preferred_element_type=jnp.float32) + bg_ref[...]
    eidx = lax.broadcasted_iota(jnp.int32, (E, T), 0).astype(jnp.float32)
    m1 = jnp.max(g, axis=0, keepdims=True)
    a1 = jnp.min(jnp.where(g == m1, eidx, float(E)), axis=0, keepdims=True)
    g2 = jnp.where(eidx == a1, -jnp.inf, g)
    m2 = jnp.max(g2, axis=0, keepdims=True)
    a2 = jnp.min(jnp.where(g2 == m2, eidx, float(E)), axis=0, keepdims=True)
    z = jnp.exp(m2 - m1)
    w1 = 1.0 / (1.0 + z)
    w1_ref[...] = w1
    w2_ref[...] = z * w1
    oh1 = (eidx == a1).astype(jnp.float32)
    oh2 = (eidx == a2).astype(jnp.float32)
    mask = oh1 + oh2
    counts = jnp.sum(mask, axis=1, keepdims=True)
    ti = lax.broadcasted_iota(jnp.int32, (T, T), 0)
    tj = lax.broadcasted_iota(jnp.int32, (T, T), 1)
    upper = (ti < tj).astype(jnp.float32)
    rank = lax.dot_general(mask, upper, (((1,), (0,)), ((), ())),
                           preferred_element_type=jnp.float32)
    mthr = lax.broadcasted_iota(jnp.int32, (E, E), 1).astype(jnp.float32) * float(B)
    nblk = jnp.sum((counts > mthr).astype(jnp.float32), axis=1,
                   keepdims=True)
    tri = (lax.broadcasted_iota(jnp.int32, (E, E), 1)
           < lax.broadcasted_iota(jnp.int32, (E, E), 0)).astype(jnp.float32)
    blk_off = lax.dot_general(tri, nblk, (((1,), (0,)), ((), ())),
                              preferred_element_type=jnp.float32)
    total = jnp.sum(nblk, axis=0, keepdims=True)
    pad_off = blk_off * float(B)
    d1_ref[...] = jnp.sum(oh1 * (rank + pad_off), axis=0,
                          keepdims=True).astype(jnp.int32)
    d2_ref[...] = jnp.sum(oh2 * (rank + pad_off), axis=0,
                          keepdims=True).astype(jnp.int32)
    bidx = lax.broadcasted_iota(jnp.int32, (1, NB), 1).astype(jnp.float32)
    bb = jnp.minimum(bidx, total - 1.0)
    incl = blk_off + nblk
    gid = jnp.sum((incl <= bb).astype(jnp.float32), axis=0, keepdims=True)
    gid_ref[...] = gid.astype(jnp.int32)
    tgb_ref[...] = bb.astype(jnp.int32)


def _routing(x, nw, wg, bg):
    return pl.pallas_call(
        _routing_body,
        out_shape=(
            jax.ShapeDtypeStruct((T, D), jnp.float32),
            jax.ShapeDtypeStruct((1, T), jnp.int32),
            jax.ShapeDtypeStruct((1, T), jnp.int32),
            jax.ShapeDtypeStruct((1, T), jnp.float32),
            jax.ShapeDtypeStruct((1, T), jnp.float32),
            jax.ShapeDtypeStruct((1, NB), jnp.int32),
            jax.ShapeDtypeStruct((1, NB), jnp.int32),
        ),
    )(x, nw, wg, bg)


def _dispatch(t, d1, d2, w1f, w2f):
    mesh = plsc.VectorSubcoreMesh(core_axis_name="c", subcore_axis_name="s")

    @functools.partial(
        pl.kernel, mesh=mesh,
        out_type=(jax.ShapeDtypeStruct((P, D), jnp.float32),
                  jax.ShapeDtypeStruct((P, 128), jnp.float32)),
        scratch_types=[pltpu.VMEM((TPW,), jnp.int32),
                       pltpu.VMEM((TPW,), jnp.int32),
                       pltpu.VMEM((TPW, D), jnp.float32),
                       pltpu.VMEM((TPW + 16,), jnp.float32),
                       pltpu.VMEM((TPW + 16,), jnp.float32),
                       pltpu.VMEM((TPW, 128), jnp.float32),
                       pltpu.VMEM((TPW, 128), jnp.float32),
                       pltpu.SemaphoreType.DMA],
    )
    def disp(t_hbm, d1_hbm, d2_hbm, w1_hbm, w2_hbm, tg_hbm, wrow_hbm,
             i1_v, i2_v, rows_v, wv1, wv2, wa_v, wb_v, sem):
        wid = lax.axis_index("s") * NC + lax.axis_index("c")
        base = wid * TPW
        pltpu.sync_copy(d1_hbm.at[pl.ds(base, TPW)], i1_v)
        pltpu.sync_copy(d2_hbm.at[pl.ds(base, TPW)], i2_v)
        pltpu.sync_copy(t_hbm.at[pl.ds(base, TPW)], rows_v)
        pltpu.sync_copy(w1_hbm.at[pl.ds(base, TPW)], wv1.at[pl.ds(0, TPW)])
        pltpu.sync_copy(w2_hbm.at[pl.ds(base, TPW)], wv2.at[pl.ds(0, TPW)])

        def fill(i, carry):
            wa_v[i, pl.ds(0, 16)] = wv1[pl.ds(i, 16)]
            wb_v[i, pl.ds(0, 16)] = wv2[pl.ds(i, 16)]
            return carry

        lax.fori_loop(0, TPW, fill, 0)
        cp1 = pltpu.async_copy(rows_v, tg_hbm.at[i1_v], sem)
        cp2 = pltpu.async_copy(rows_v, tg_hbm.at[i2_v], sem)
        cp3 = pltpu.async_copy(wa_v, wrow_hbm.at[i1_v], sem)
        cp4 = pltpu.async_copy(wb_v, wrow_hbm.at[i2_v], sem)
        cp1.wait()
        cp2.wait()
        cp3.wait()
        cp4.wait()

    return disp(t, d1, d2, w1f, w2f)


def _mm_body(gid_ref, tgb_ref, tg_ref, wrow_ref, w1g_ref, w1u_ref, w2_ref,
             y_ref, acc_ref):
    b = pl.program_id(0)
    f = pl.program_id(1)

    @pl.when(tgb_ref[b] == b)
    def _():
        tgb16 = tg_ref[...].astype(jnp.bfloat16)
        hg = lax.dot_general(tgb16, w1g_ref[0], (((1,), (1,)), ((), ())),
                             preferred_element_type=jnp.float32)
        hu = lax.dot_general(tgb16, w1u_ref[0], (((1,), (1,)), ((), ())),
                             preferred_element_type=jnp.float32)
        h = hg * (1.0 / (1.0 + jnp.exp(-hg))) * hu

        yf = lax.dot_general(h.astype(jnp.bfloat16), w2_ref[0],
                             (((1,), (1,)), ((), ())),
                             preferred_element_type=jnp.float32)

        @pl.when(f == 0)
        def _():
            acc_ref[...] = yf

        @pl.when(f > 0)
        def _():
            acc_ref[...] += yf

        @pl.when(f == NF - 1)
        def _():
            y_ref[...] = acc_ref[...] * wrow_ref[:, 0:1]


def _mm(gid, tgb, tg, wrow, w1g, w1u, w2):
    grid_spec = pltpu.PrefetchScalarGridSpec(
        num_scalar_prefetch=2,
        grid=(NB, NF),
        in_specs=[
            pl.BlockSpec((B, D), lambda b, f, gid, tgb: (tgb[b], 0)),
            pl.BlockSpec((B, 128), lambda b, f, gid, tgb: (tgb[b], 0)),
            pl.BlockSpec((1, FT, D), lambda b, f, gid, tgb: (gid[b], f, 0)),
            pl.BlockSpec((1, FT, D), lambda b, f, gid, tgb: (gid[b], f, 0)),
            pl.BlockSpec((1, D, FT), lambda b, f, gid, tgb: (gid[b], 0, f)),
        ],
        out_specs=pl.BlockSpec((B, D), lambda b, f, gid, tgb: (tgb[b], 0)),
        scratch_shapes=[pltpu.VMEM((B, D), jnp.float32)],
    )
    return pl.pallas_call(
        _mm_body,
        grid_spec=grid_spec,
        out_shape=jax.ShapeDtypeStruct((P, D), jnp.float32),
    )(gid, tgb, tg, wrow, w1g, w1u, w2)


def _combine(x, y, d1, d2):
    mesh = plsc.VectorSubcoreMesh(core_axis_name="c", subcore_axis_name="s")

    @functools.partial(
        pl.kernel, mesh=mesh,
        out_type=jax.ShapeDtypeStruct((T, D), jnp.float32),
        scratch_types=[pltpu.VMEM((CH,), jnp.int32),
                       pltpu.VMEM((CH,), jnp.int32),
                       pltpu.VMEM((CH, D), jnp.float32),
                       pltpu.VMEM((CH, D), jnp.float32),
                       pltpu.VMEM((CH, D), jnp.float32),
                       pltpu.SemaphoreType.DMA],
    )
    def comb(x_hbm, y_hbm, d1_hbm, d2_hbm, out_hbm,
             i1_v, i2_v, r1_v, r2_v, xv, sem):
        wid = lax.axis_index("s") * NC + lax.axis_index("c")
        for c in range(TPW // CH):
            base = wid * TPW + c * CH
            pltpu.sync_copy(d1_hbm.at[pl.ds(base, CH)], i1_v)
            pltpu.sync_copy(d2_hbm.at[pl.ds(base, CH)], i2_v)
            pltpu.sync_copy(x_hbm.at[pl.ds(base, CH)], xv)
            cp1 = pltpu.async_copy(y_hbm.at[i1_v], r1_v, sem)
            cp2 = pltpu.async_copy(y_hbm.at[i2_v], r2_v, sem)
            cp1.wait()
            cp2.wait()

            def tok(i, carry):
                for j in range(D // 16):
                    sl = pl.ds(j * 16, 16)
                    xv[i, sl] = xv[i, sl] + r1_v[i, sl] + r2_v[i, sl]
                return carry

            lax.fori_loop(0, CH, tok, 0)
            pltpu.sync_copy(xv, out_hbm.at[pl.ds(base, CH)])

    return comb(x, y, d1, d2)


def kernel(x, norm_w, Wg, bg, W1g, W1u, W2):
    t, d1, d2, w1, w2, gid, tgb = _routing(
        x, norm_w.reshape(1, D), Wg, bg.reshape(E, 1))
    d1f = d1.reshape(T)
    d2f = d2.reshape(T)
    tg, wrow = _dispatch(t, d1f, d2f, w1.reshape(T), w2.reshape(T))
    y = _mm(gid.reshape(NB), tgb.reshape(NB), tg, wrow, W1g, W1u, W2)
    return _combine(x, y, d1f, d2f)


# speedup vs baseline: 2.1157x; 1.1138x over previous
import functools

import jax
import jax.numpy as jnp
from jax import lax
from jax.experimental import pallas as pl
from jax.experimental.pallas import tpu as pltpu
from jax.experimental.pallas import tpu_sc as plsc

T = 2048
D = 1024
F = 2048
E = 8
EPS = 1e-6

B = 576
NB = 7 + (T * 2 - 7 + B - 1) // B
P = NB * B
FT = 1024
NF = F // FT

NC = 2
NS = 16
NW = NC * NS
TPW = T // NW
CH = 16


def _routing_body(x_ref, nw_ref, wg_ref, bg_ref,
                  t_ref, d1_ref, d2_ref, w1_ref, w2_ref, gid_ref, tgb_ref):
    x = x_ref[...]
    r = lax.rsqrt(jnp.mean(x * x, axis=1, keepdims=True) + EPS)
    t = x * r * nw_ref[...]
    t_ref[...] = t
    g = lax.dot_general(wg_ref[...], t, (((1,), (1,)), ((), ())),
                        preferred_element_type=jnp.float32) + bg_ref[...]
    eidx = lax.broadcasted_iota(jnp.int32, (E, T), 0).astype(jnp.float32)
    m1 = jnp.max(g, axis=0, keepdims=True)
    a1 = jnp.min(jnp.where(g == m1, eidx, float(E)), axis=0, keepdims=True)
    g2 = jnp.where(eidx == a1, -jnp.inf, g)
    m2 = jnp.max(g2, axis=0, keepdims=True)
    a2 = jnp.min(jnp.where(g2 == m2, eidx, float(E)), axis=0, keepdims=True)
    z = jnp.exp(m2 - m1)
    w1 = 1.0 / (1.0 + z)
    w1_ref[...] = w1
    w2_ref[...] = z * w1
    oh1 = (eidx == a1).astype(jnp.float32)
    oh2 = (eidx == a2).astype(jnp.float32)
    mask = oh1 + oh2
    counts = jnp.sum(mask, axis=1, keepdims=True)
    ti = lax.broadcasted_iota(jnp.int32, (T, T), 0)
    tj = lax.broadcasted_iota(jnp.int32, (T, T), 1)
    upper = (ti < tj).astype(jnp.float32)
    rank = lax.dot_general(mask, upper, (((1,), (0,)), ((), ())),
                           preferred_element_type=jnp.float32)
    mthr = lax.broadcasted_iota(jnp.int32, (E, E), 1).astype(jnp.float32) * float(B)
    nblk = jnp.sum((counts > mthr).astype(jnp.float32), axis=1,
                   keepdims=True)
    tri = (lax.broadcasted_iota(jnp.int32, (E, E), 1)
           < lax.broadcasted_iota(jnp.int32, (E, E), 0)).astype(jnp.float32)
    blk_off = lax.dot_general(tri, nblk, (((1,), (0,)), ((), ())),
                              preferred_element_type=jnp.float32)
    total = jnp.sum(nblk, axis=0, keepdims=True)
    pad_off = blk_off * float(B)
    d1_ref[...] = jnp.sum(oh1 * (rank + pad_off), axis=0,
                          keepdims=True).astype(jnp.int32)
    d2_ref[...] = jnp.sum(oh2 * (rank + pad_off), axis=0,
                          keepdims=True).astype(jnp.int32)
    bidx = lax.broadcasted_iota(jnp.int32, (1, NB), 1).astype(jnp.float32)
    bb = jnp.minimum(bidx, total - 1.0)
    incl = blk_off + nblk
    gid = jnp.sum((incl <= bb).astype(jnp.float32), axis=0, keepdims=True)
    gid_ref[...] = gid.astype(jnp.int32)
    tgb_ref[...] = bb.astype(jnp.int32)


def _routing(x, nw, wg, bg):
    return pl.pallas_call(
        _routing_body,
        out_shape=(
            jax.ShapeDtypeStruct((T, D), jnp.float32),
            jax.ShapeDtypeStruct((1, T), jnp.int32),
            jax.ShapeDtypeStruct((1, T), jnp.int32),
            jax.ShapeDtypeStruct((1, T), jnp.float32),
            jax.ShapeDtypeStruct((1, T), jnp.float32),
            jax.ShapeDtypeStruct((1, NB), jnp.int32),
            jax.ShapeDtypeStruct((1, NB), jnp.int32),
        ),
    )(x, nw, wg, bg)


def _dispatch(t, d1, d2, w1f, w2f):
    mesh = plsc.VectorSubcoreMesh(core_axis_name="c", subcore_axis_name="s")

    @functools.partial(
        pl.kernel, mesh=mesh,
        out_type=(jax.ShapeDtypeStruct((P, D), jnp.float32),
                  jax.ShapeDtypeStruct((P, 128), jnp.float32)),
        scratch_types=[pltpu.VMEM((TPW,), jnp.int32),
                       pltpu.VMEM((TPW,), jnp.int32),
                       pltpu.VMEM((TPW, D), jnp.float32),
                       pltpu.VMEM((TPW + 16,), jnp.float32),
                       pltpu.VMEM((TPW + 16,), jnp.float32),
                       pltpu.VMEM((TPW, 128), jnp.float32),
                       pltpu.VMEM((TPW, 128), jnp.float32),
                       pltpu.SemaphoreType.DMA],
    )
    def disp(t_hbm, d1_hbm, d2_hbm, w1_hbm, w2_hbm, tg_hbm, wrow_hbm,
             i1_v, i2_v, rows_v, wv1, wv2, wa_v, wb_v, sem):
        wid = lax.axis_index("s") * NC + lax.axis_index("c")
        base = wid * TPW
        pltpu.sync_copy(d1_hbm.at[pl.ds(base, TPW)], i1_v)
        pltpu.sync_copy(d2_hbm.at[pl.ds(base, TPW)], i2_v)
        pltpu.sync_copy(t_hbm.at[pl.ds(base, TPW)], rows_v)
        pltpu.sync_copy(w1_hbm.at[pl.ds(base, TPW)], wv1.at[pl.ds(0, TPW)])
        pltpu.sync_copy(w2_hbm.at[pl.ds(base, TPW)], wv2.at[pl.ds(0, TPW)])

        def fill(i, carry):
            wa_v[i, pl.ds(0, 16)] = wv1[pl.ds(i, 16)]
            wb_v[i, pl.ds(0, 16)] = wv2[pl.ds(i, 16)]
            return carry

        lax.fori_loop(0, TPW, fill, 0)
        cp1 = pltpu.async_copy(rows_v, tg_hbm.at[i1_v], sem)
        cp2 = pltpu.async_copy(rows_v, tg_hbm.at[i2_v], sem)
        cp3 = pltpu.async_copy(wa_v, wrow_hbm.at[i1_v], sem)
        cp4 = pltpu.async_copy(wb_v, wrow_hbm.at[i2_v], sem)
        cp1.wait()
        cp2.wait()
        cp3.wait()
        cp4.wait()

    return disp(t, d1, d2, w1f, w2f)


def _mm_body(gid_ref, tgb_ref, tg_ref, wrow_ref, w1g_ref, w1u_ref, w2_ref,
             y_ref, acc_ref):
    b = pl.program_id(0)
    f = pl.program_id(1)

    @pl.when(tgb_ref[b] == b)
    def _():
        tgb16 = tg_ref[...].astype(jnp.bfloat16)
        hg = lax.dot_general(tgb16, w1g_ref[0], (((1,), (1,)), ((), ())),
                             preferred_element_type=jnp.float32)
        hu = lax.dot_general(tgb16, w1u_ref[0], (((1,), (1,)), ((), ())),
                             preferred_element_type=jnp.float32)
        h = hg * (1.0 / (1.0 + jnp.exp(-hg))) * hu

        yf = lax.dot_general(h.astype(jnp.bfloat16), w2_ref[0],
                             (((1,), (1,)), ((), ())),
                             preferred_element_type=jnp.float32)

        @pl.when(f == 0)
        def _():
            acc_ref[...] = yf

        @pl.when(f > 0)
        def _():
            acc_ref[...] += yf

        @pl.when(f == NF - 1)
        def _():
            y_ref[...] = acc_ref[...] * wrow_ref[:, 0:1]


def _mm(gid, tgb, tg, wrow, w1g, w1u, w2):
    grid_spec = pltpu.PrefetchScalarGridSpec(
        num_scalar_prefetch=2,
        grid=(NB, NF),
        in_specs=[
            pl.BlockSpec((B, D), lambda b, f, gid, tgb: (tgb[b], 0)),
            pl.BlockSpec((B, 128), lambda b, f, gid, tgb: (tgb[b], 0)),
            pl.BlockSpec((1, FT, D), lambda b, f, gid, tgb: (gid[b], f, 0)),
            pl.BlockSpec((1, FT, D), lambda b, f, gid, tgb: (gid[b], f, 0)),
            pl.BlockSpec((1, D, FT), lambda b, f, gid, tgb: (gid[b], 0, f)),
        ],
        out_specs=pl.BlockSpec((B, D), lambda b, f, gid, tgb: (tgb[b], 0)),
        scratch_shapes=[pltpu.VMEM((B, D), jnp.float32)],
    )
    return pl.pallas_call(
        _mm_body,
        grid_spec=grid_spec,
        out_shape=jax.ShapeDtypeStruct((P, D), jnp.float32),
    )(gid, tgb, tg, wrow, w1g, w1u, w2)


def _combine(x, y, d1, d2):
    mesh = plsc.VectorSubcoreMesh(core_axis_name="c", subcore_axis_name="s")
    nch = TPW // CH

    @functools.partial(
        pl.kernel, mesh=mesh,
        out_type=jax.ShapeDtypeStruct((T, D), jnp.float32),
        scratch_types=[pltpu.VMEM((TPW,), jnp.int32),
                       pltpu.VMEM((TPW,), jnp.int32),
                       pltpu.VMEM((2, CH, D), jnp.float32),
                       pltpu.VMEM((2, CH, D), jnp.float32),
                       pltpu.VMEM((2, CH, D), jnp.float32),
                       pltpu.SemaphoreType.DMA,
                       pltpu.SemaphoreType.DMA,
                       pltpu.SemaphoreType.DMA,
                       pltpu.SemaphoreType.DMA],
    )
    def comb(x_hbm, y_hbm, d1_hbm, d2_hbm, out_hbm,
             i1_v, i2_v, r1_v, r2_v, xv, sin0, sin1, sout0, sout1):
        wid = lax.axis_index("s") * NC + lax.axis_index("c")
        base = wid * TPW
        pltpu.sync_copy(d1_hbm.at[pl.ds(base, TPW)], i1_v)
        pltpu.sync_copy(d2_hbm.at[pl.ds(base, TPW)], i2_v)
        sin = (sin0, sin1)
        sout = (sout0, sout1)
        inflight = [None, None]
        outflight = [None, None]

        def issue(c):
            s = c % 2
            cps = (pltpu.async_copy(y_hbm.at[i1_v.at[pl.ds(c * CH, CH)]],
                                    r1_v.at[s], sin[s]),
                   pltpu.async_copy(y_hbm.at[i2_v.at[pl.ds(c * CH, CH)]],
                                    r2_v.at[s], sin[s]),
                   pltpu.async_copy(x_hbm.at[pl.ds(base + c * CH, CH)],
                                    xv.at[s], sin[s]))
            inflight[s] = cps

        issue(0)
        for c in range(nch):
            s = c % 2
            if c + 1 < nch:
                if outflight[(c + 1) % 2] is not None:
                    outflight[(c + 1) % 2].wait()
                    outflight[(c + 1) % 2] = None
                issue(c + 1)
            for cp in inflight[s]:
                cp.wait()

            def tok(i, carry):
                for j in range(D // 16):
                    sl = pl.ds(j * 16, 16)
                    xv[s, i, sl] = xv[s, i, sl] + r1_v[s, i, sl] + r2_v[s, i, sl]
                return carry

            lax.fori_loop(0, CH, tok, 0)
            outflight[s] = pltpu.async_copy(
                xv.at[s], out_hbm.at[pl.ds(base + c * CH, CH)], sout[s])
        for s in range(2):
            if outflight[s] is not None:
                outflight[s].wait()

    return comb(x, y, d1, d2)


def kernel(x, norm_w, Wg, bg, W1g, W1u, W2):
    t, d1, d2, w1, w2, gid, tgb = _routing(
        x, norm_w.reshape(1, D), Wg, bg.reshape(E, 1))
    d1f = d1.reshape(T)
    d2f = d2.reshape(T)
    tg, wrow = _dispatch(t, d1f, d2f, w1.reshape(T), w2.reshape(T))
    y = _mm(gid.reshape(NB), tgb.reshape(NB), tg, wrow, W1g, W1u, W2)
    return _combine(x, y, d1f, d2f)
